# Initial kernel scaffold; baseline (speedup 1.0000x reference)
import jax
import jax.numpy as jnp
from jax.experimental import pallas as pl


def kernel(x, conv_w, conv_b, head_w, head_b):
    raise NotImplementedError("write your pallas kernel here")


# speedup vs baseline: 49.9015x; 49.9015x over previous
import jax
import jax.numpy as jnp
from jax.experimental import pallas as pl
from jax.experimental.pallas import tpu as pltpu

NBINS = 64
COUT = 32
K = 3
H = 384
HO = H - K + 1


def _hist_head_kernel(x_ref, w_ref, b_ref, wt_ref, hb_ref, out_ref):
    c = pl.program_id(1)
    acc = jnp.zeros((HO, HO), dtype=jnp.float32)
    for di in range(K):
        for dj in range(K):
            acc = acc + w_ref[c, di * K + dj] * x_ref[0, di:di + HO, dj:dj + HO]
    y = jnp.maximum(acc + b_ref[c], 0.0)
    lo = jnp.min(y)
    hi = jnp.max(y)
    same = hi == lo
    lo = jnp.where(same, lo - 1.0, lo)
    hi = jnp.where(same, hi + 1.0, hi)
    scale = NBINS / (hi - lo)
    idx = jnp.floor((y - lo) * scale).astype(jnp.int32)
    idx = jnp.clip(idx, 0, NBINS - 1)
    cnts = jnp.stack(
        [jnp.sum((idx == k).astype(jnp.float32)) for k in range(NBINS)]
    )
    h = cnts.reshape(1, NBINS)
    contrib = jnp.dot(h, wt_ref[0], preferred_element_type=jnp.float32)

    @pl.when(c == 0)
    def _():
        out_ref[0] = contrib + hb_ref[...].reshape(1, -1)

    @pl.when(c > 0)
    def _():
        out_ref[0] = out_ref[0] + contrib


def kernel(x, conv_w, conv_b, head_w, head_b):
    B = x.shape[0]
    xs = x.reshape(B, H, H)
    wf = conv_w.reshape(COUT, K * K)
    wt = head_w.T.reshape(COUT, NBINS, head_w.shape[0])
    out = pl.pallas_call(
        _hist_head_kernel,
        grid=(B, COUT),
        in_specs=[
            pl.BlockSpec((1, H, H), lambda b, c: (b, 0, 0)),
            pl.BlockSpec(memory_space=pltpu.SMEM),
            pl.BlockSpec(memory_space=pltpu.SMEM),
            pl.BlockSpec((1, NBINS, head_w.shape[0]), lambda b, c: (c, 0, 0)),
            pl.BlockSpec((head_w.shape[0],), lambda b, c: (0,)),
        ],
        out_specs=pl.BlockSpec((1, 1, head_w.shape[0]), lambda b, c: (b, 0, 0)),
        out_shape=jax.ShapeDtypeStruct((B, 1, head_w.shape[0]), jnp.float32),
    )(xs, wf, conv_b, wt, head_b)
    return out.reshape(B, head_w.shape[0])
